# Initial kernel scaffold; baseline (speedup 1.0000x reference)
import jax
import jax.numpy as jnp
from jax.experimental import pallas as pl


def kernel(x):
    raise NotImplementedError("write your pallas kernel here")


# speedup vs baseline: 2.6705x; 2.6705x over previous
import functools

import jax
import jax.numpy as jnp
from jax.experimental import pallas as pl
from jax.experimental.pallas import tpu as pltpu


def _ce_reshape(x, k, d):
    n, lanes = x.shape
    g = n // (2 * d)
    x4 = x.reshape(g, 2, d, lanes)
    lo = x4[:, 0]
    hi = x4[:, 1]
    mn = jnp.minimum(lo, hi)
    mx = jnp.maximum(lo, hi)
    obit = (jax.lax.broadcasted_iota(jnp.int32, (g, 1, 1), 0) * (2 * d)) >> (k + 1)
    desc = (obit & 1) == 1
    new_lo = jnp.where(desc, mx, mn)
    new_hi = jnp.where(desc, mn, mx)
    return jnp.concatenate(
        [new_lo.reshape(g, 1, d, lanes), new_hi.reshape(g, 1, d, lanes)], axis=1
    ).reshape(n, lanes)


def _ce_roll(x, k, d):
    n, lanes = x.shape
    i = jax.lax.broadcasted_iota(jnp.int32, (n, 1), 0)
    up = pltpu.roll(x, n - d, axis=0)
    down = pltpu.roll(x, d, axis=0)
    low_half = (i & d) == 0
    partner = jnp.where(low_half, up, down)
    desc = (i >> (k + 1)) & 1 == 1
    keep_min = low_half != desc
    return jnp.where(keep_min, jnp.minimum(x, partner), jnp.maximum(x, partner))


def _bitonic_body(x_ref, o_ref):
    x = x_ref[0]
    n = x.shape[0]
    log_n = n.bit_length() - 1
    for k in range(log_n):
        for j in range(k, -1, -1):
            d = 1 << j
            if d >= 8:
                x = _ce_reshape(x, k, d)
            else:
                x = _ce_roll(x, k, d)
    o_ref[0] = x


@functools.partial(jax.jit, static_argnames=("block_lanes", "interpret"))
def _sort_mid(x, block_lanes=256, interpret=False):
    b, n, m = x.shape
    grid = (b, m // block_lanes)
    spec = pl.BlockSpec((1, n, block_lanes), lambda i, j: (i, 0, j))
    return pl.pallas_call(
        _bitonic_body,
        grid=grid,
        in_specs=[spec],
        out_specs=spec,
        out_shape=jax.ShapeDtypeStruct(x.shape, x.dtype),
        interpret=interpret,
    )(x)


def kernel(x):
    return _sort_mid(x)
